# Initial kernel scaffold; baseline (speedup 1.0000x reference)
import jax
import jax.numpy as jnp
from jax.experimental import pallas as pl


def kernel(h, edge_index, Wm0, bm0, Wm1, bm1, Wu0, bu0, Wu1, bu1, W_ih, b_ih, W_hh, b_hh, We1, be1, We2, be2):
    raise NotImplementedError("write your pallas kernel here")


# speedup vs baseline: 2.2090x; 2.2090x over previous
import functools

import jax
import jax.numpy as jnp
from jax import lax
from jax.experimental import pallas as pl
from jax.experimental.pallas import tpu as pltpu
from jax.experimental.pallas import tpu_sc as plsc

N = 10000
D = 128
E = 320000

NPAD = 10240
NW = 32
CHUNK = 128
NCH = 79
EPT = NCH * CHUNK
EPAD = NW * EPT
ROWS_PER_TILE = NPAD // 16

BLK = 2048
EBLK = 4096


@functools.lru_cache(maxsize=None)
def _sc_kernels():
    mesh = plsc.VectorSubcoreMesh(core_axis_name="c", subcore_axis_name="s")

    @functools.partial(
        pl.kernel,
        out_type=jax.ShapeDtypeStruct((2, NPAD, D), jnp.float32),
        mesh=mesh,
        scratch_types=[
            pltpu.VMEM((CHUNK,), jnp.int32),
            pltpu.VMEM((CHUNK,), jnp.int32),
            pltpu.VMEM((CHUNK, D), jnp.float32),
            pltpu.VMEM_SHARED((NPAD, D), jnp.float32),
            pltpu.SemaphoreType.DMA,
        ],
    )
    def _sc_scatter_add(m_hbm, srcw_hbm, dstw_hbm, zeros_hbm, parts_hbm,
                        idx_s, idx_d, rows, agg, sem):
        c = lax.axis_index("c")
        s = lax.axis_index("s")
        wid = c * 16 + s
        pltpu.sync_copy(zeros_hbm,
                        agg.at[pl.ds(s * ROWS_PER_TILE, ROWS_PER_TILE)])
        plsc.subcore_barrier()

        def body(j, carry):
            pltpu.sync_copy(srcw_hbm.at[wid, j], idx_s)
            pltpu.sync_copy(dstw_hbm.at[wid, j], idx_d)
            pltpu.async_copy(m_hbm.at[idx_s], rows, sem).wait()
            pltpu.sync_copy(rows, agg.at[idx_d], add=True)
            return carry

        lax.fori_loop(0, NCH, body, 0)
        plsc.subcore_barrier()
        pltpu.sync_copy(agg.at[pl.ds(s * ROWS_PER_TILE, ROWS_PER_TILE)],
                        parts_hbm.at[c, pl.ds(s * ROWS_PER_TILE, ROWS_PER_TILE)])

    @functools.partial(
        pl.kernel,
        out_type=(jax.ShapeDtypeStruct((EPAD, D), jnp.float32),
                  jax.ShapeDtypeStruct((EPAD, D), jnp.float32)),
        mesh=mesh,
        scratch_types=[
            pltpu.VMEM((CHUNK,), jnp.int32),
            pltpu.VMEM((CHUNK,), jnp.int32),
            pltpu.VMEM((CHUNK, D), jnp.float32),
            pltpu.VMEM((CHUNK, D), jnp.float32),
            pltpu.SemaphoreType.DMA,
            pltpu.SemaphoreType.DMA,
        ],
    )
    def _sc_pair_gather(h_hbm, srcw_hbm, dstw_hbm, hu_hbm, hv_hbm,
                        idx_u, idx_v, rows_u, rows_v, sem_u, sem_v):
        c = lax.axis_index("c")
        s = lax.axis_index("s")
        wid = c * 16 + s
        base = wid * EPT

        def body(j, carry):
            off = base + j * CHUNK
            pltpu.sync_copy(srcw_hbm.at[wid, j], idx_u)
            pltpu.sync_copy(dstw_hbm.at[wid, j], idx_v)
            cu = pltpu.async_copy(h_hbm.at[idx_u], rows_u, sem_u)
            cv = pltpu.async_copy(h_hbm.at[idx_v], rows_v, sem_v)
            cu.wait()
            cv.wait()
            pltpu.sync_copy(rows_u, hu_hbm.at[pl.ds(off, CHUNK)])
            pltpu.sync_copy(rows_v, hv_hbm.at[pl.ds(off, CHUNK)])
            return carry

        lax.fori_loop(0, NCH, body, 0)

    return _sc_scatter_add, _sc_pair_gather


def _msg_body(h_ref, wt_ref, b_ref, o_ref):
    i = pl.program_id(0)
    y = jnp.dot(h_ref[...], wt_ref[...], preferred_element_type=jnp.float32)
    y = jnp.maximum(y + b_ref[...], 0.0)
    rows = lax.broadcasted_iota(jnp.int32, y.shape, 0) + i * BLK
    o_ref[...] = jnp.where(rows < N, y, 0.0)


def _msg(h_pad, WmT, bm):
    return pl.pallas_call(
        _msg_body,
        grid=(NPAD // BLK,),
        in_specs=[pl.BlockSpec((BLK, D), lambda i: (i, 0)),
                  pl.BlockSpec((D, D), lambda i: (0, 0)),
                  pl.BlockSpec((1, D), lambda i: (0, 0))],
        out_specs=pl.BlockSpec((BLK, D), lambda i: (i, 0)),
        out_shape=jax.ShapeDtypeStruct((NPAD, D), jnp.float32),
    )(h_pad, WmT, bm)


def _gru_math(p0, p1, hb, WuT, bu, WihT, bih, WhhT, bhh):
    agg = p0 + p1
    msg = jnp.dot(agg, WuT, preferred_element_type=jnp.float32) + bu
    msg = jnp.maximum(msg, 0.0)
    gi = jnp.dot(msg, WihT, preferred_element_type=jnp.float32) + bih
    gh = jnp.dot(hb, WhhT, preferred_element_type=jnp.float32) + bhh
    r = jax.nn.sigmoid(gi[:, :D] + gh[:, :D])
    z = jax.nn.sigmoid(gi[:, D:2 * D] + gh[:, D:2 * D])
    n = jnp.tanh(gi[:, 2 * D:] + r * gh[:, 2 * D:])
    return (1.0 - z) * n + z * hb


def _upd_m_body(parts_ref, h_ref, WuT, bu, WihT, bih, WhhT, bhh, WmT, bm,
                h_out, m_out):
    i = pl.program_id(0)
    hn = _gru_math(parts_ref[0], parts_ref[1], h_ref[...], WuT[...], bu[...],
                   WihT[...], bih[...], WhhT[...], bhh[...])
    h_out[...] = hn
    y = jnp.dot(hn, WmT[...], preferred_element_type=jnp.float32)
    y = jnp.maximum(y + bm[...], 0.0)
    rows = lax.broadcasted_iota(jnp.int32, y.shape, 0) + i * BLK
    m_out[...] = jnp.where(rows < N, y, 0.0)


def _upd_m(parts, h_pad, WuT, bu, WihT, bih, WhhT, bhh, WmT, bm):
    full = lambda shape: pl.BlockSpec(shape, lambda i: tuple(0 for _ in shape))
    return pl.pallas_call(
        _upd_m_body,
        grid=(NPAD // BLK,),
        in_specs=[pl.BlockSpec((2, BLK, D), lambda i: (0, i, 0)),
                  pl.BlockSpec((BLK, D), lambda i: (i, 0)),
                  full((D, D)), full((1, D)),
                  full((D, 3 * D)), full((1, 3 * D)),
                  full((D, 3 * D)), full((1, 3 * D)),
                  full((D, D)), full((1, D))],
        out_specs=(pl.BlockSpec((BLK, D), lambda i: (i, 0)),
                   pl.BlockSpec((BLK, D), lambda i: (i, 0))),
        out_shape=(jax.ShapeDtypeStruct((NPAD, D), jnp.float32),
                   jax.ShapeDtypeStruct((NPAD, D), jnp.float32)),
    )(parts, h_pad, WuT, bu, WihT, bih, WhhT, bhh, WmT, bm)


def _upd_body(parts_ref, h_ref, WuT, bu, WihT, bih, WhhT, bhh, h_out):
    h_out[...] = _gru_math(parts_ref[0], parts_ref[1], h_ref[...], WuT[...],
                           bu[...], WihT[...], bih[...], WhhT[...], bhh[...])


def _upd(parts, h_pad, WuT, bu, WihT, bih, WhhT, bhh):
    full = lambda shape: pl.BlockSpec(shape, lambda i: tuple(0 for _ in shape))
    return pl.pallas_call(
        _upd_body,
        grid=(NPAD // BLK,),
        in_specs=[pl.BlockSpec((2, BLK, D), lambda i: (0, i, 0)),
                  pl.BlockSpec((BLK, D), lambda i: (i, 0)),
                  full((D, D)), full((1, D)),
                  full((D, 3 * D)), full((1, 3 * D)),
                  full((D, 3 * D)), full((1, 3 * D))],
        out_specs=pl.BlockSpec((BLK, D), lambda i: (i, 0)),
        out_shape=jax.ShapeDtypeStruct((NPAD, D), jnp.float32),
    )(parts, h_pad, WuT, bu, WihT, bih, WhhT, bhh)


def _score_body(hu_ref, hv_ref, W1T_ref, b1_ref, w2_ref, b2_ref, o_ref):
    u = hu_ref[...]
    v = hv_ref[...]
    W1T = W1T_ref[...]
    hid = jnp.dot(u, W1T[:D], preferred_element_type=jnp.float32)
    hid += jnp.dot(v, W1T[D:2 * D], preferred_element_type=jnp.float32)
    hid += jnp.dot(jnp.abs(u - v), W1T[2 * D:3 * D],
                   preferred_element_type=jnp.float32)
    hid += jnp.dot(u * v, W1T[3 * D:], preferred_element_type=jnp.float32)
    hid = jnp.maximum(hid + b1_ref[...], 0.0)
    o_ref[...] = jnp.sum(hid * w2_ref[...] + b2_ref[...], axis=1)


def _score(hu, hv, W1T, b1, w2, b2row):
    full = lambda shape: pl.BlockSpec(shape, lambda i: tuple(0 for _ in shape))
    return pl.pallas_call(
        _score_body,
        grid=(EPAD // EBLK,),
        in_specs=[pl.BlockSpec((EBLK, D), lambda i: (i, 0)),
                  pl.BlockSpec((EBLK, D), lambda i: (i, 0)),
                  full((4 * D, D)), full((1, D)), full((1, D)), full((1, D))],
        out_specs=pl.BlockSpec((EBLK,), lambda i: (i,)),
        out_shape=jax.ShapeDtypeStruct((EPAD,), jnp.float32),
    )(hu, hv, W1T, b1, w2, b2row)


def kernel(h, edge_index, Wm0, bm0, Wm1, bm1, Wu0, bu0, Wu1, bu1,
           W_ih, b_ih, W_hh, b_hh, We1, be1, We2, be2):
    src = edge_index[0]
    dst = edge_index[1]
    padi = jnp.full((EPAD - E,), N, jnp.int32)
    srcw = jnp.concatenate([src, padi]).reshape(NW, NCH, CHUNK)
    dstw = jnp.concatenate([dst, padi]).reshape(NW, NCH, CHUNK)
    h0 = jnp.pad(h, ((0, NPAD - N), (0, 0)))
    zrows = jnp.zeros((ROWS_PER_TILE, D), jnp.float32)

    _sc_scatter_add, _sc_pair_gather = _sc_kernels()

    m0 = _msg(h0, Wm0.T, bm0[None])
    parts0 = _sc_scatter_add(m0, srcw, dstw, zrows)
    h1, m1 = _upd_m(parts0, h0, Wu0.T, bu0[None], W_ih.T, b_ih[None],
                    W_hh.T, b_hh[None], Wm1.T, bm1[None])
    parts1 = _sc_scatter_add(m1, srcw, dstw, zrows)
    h2 = _upd(parts1, h1, Wu1.T, bu1[None], W_ih.T, b_ih[None],
              W_hh.T, b_hh[None])
    hu, hv = _sc_pair_gather(h2, srcw, dstw)
    b2row = jnp.full((1, D), be2[0] / D, jnp.float32)
    sc = _score(hu, hv, We1.T, be1[None], We2, b2row)
    return sc[:E]
